# Initial kernel scaffold; baseline (speedup 1.0000x reference)
import jax
import jax.numpy as jnp
from jax.experimental import pallas as pl


def kernel(x, edge_index, W_l, b_l, W_r):
    raise NotImplementedError("write your pallas kernel here")


# speedup vs baseline: 3.3713x; 3.3713x over previous
import functools

import jax
import jax.numpy as jnp
from jax import lax
from jax.experimental import pallas as pl
from jax.experimental.pallas import tpu as pltpu
from jax.experimental.pallas import tpu_sc as plsc

N = 10000
E = 320000
D = 128

NC = 2
NS = 16
NW = NC * NS
K = 128
NP = 10112
EP = 327680
NCH = EP // (NW * K)
RPT = NP // NS


CW = 8


def _sc_body(x_hbm, src_hbm, dst_hbm, sums_hbm,
             src_v, dst_v, rows_v, acc_sh, sem):
    cid = lax.axis_index("c")
    sid = lax.axis_index("s")
    wid = cid * NS + sid

    pltpu.sync_copy(src_hbm.at[wid], src_v)
    pltpu.sync_copy(dst_hbm.at[wid], dst_v)

    zeros16 = jnp.zeros((16,), jnp.float32)

    def fill_rows(i, c):
        def inner(j, c2):
            rows_v[i, pl.ds(j * 16, 16)] = zeros16
            return c2
        return lax.fori_loop(0, D // 16, inner, c)

    lax.fori_loop(0, K, fill_rows, 0)

    base = sid * RPT
    for t in range(4):
        pltpu.sync_copy(rows_v, acc_sh.at[pl.ds(base + t * K, K)])
    pltpu.sync_copy(rows_v.at[pl.ds(0, RPT - 4 * K)],
                    acc_sh.at[pl.ds(base + 4 * K, RPT - 4 * K)])
    plsc.subcore_barrier()

    def chunk(j, c):
        pltpu.async_copy(x_hbm.at[src_v.at[j]], rows_v, sem).wait()
        pltpu.sync_copy(rows_v, acc_sh.at[dst_v.at[j]], add=True)
        return c

    lax.fori_loop(0, NCH, chunk, 0)

    plsc.subcore_barrier()

    pltpu.sync_copy(acc_sh.at[pl.ds(base, RPT)],
                    sums_hbm.at[cid, pl.ds(base, RPT)])


_sc_aggregate = functools.partial(
    pl.kernel,
    out_type=jax.ShapeDtypeStruct((NC, NP, D), jnp.float32),
    mesh=plsc.VectorSubcoreMesh(core_axis_name="c", subcore_axis_name="s"),
    compiler_params=pltpu.CompilerParams(use_tc_tiling_on_sc=False),
    scratch_types=(
        pltpu.VMEM((NCH, K), jnp.int32),
        pltpu.VMEM((NCH, K), jnp.int32),
        pltpu.VMEM((K, D), jnp.float32),
        pltpu.VMEM_SHARED((NP, D), jnp.float32),
        pltpu.SemaphoreType.DMA,
    ),
)(_sc_body)


def _sc_count_body(dst_hbm, ones_hbm, zc_hbm, cnts_hbm,
                   dst_v, ones_v, cnt_sh):
    cid = lax.axis_index("c")
    sid = lax.axis_index("s")
    wid = cid * NS + sid

    pltpu.sync_copy(dst_hbm.at[wid], dst_v)
    pltpu.sync_copy(ones_hbm, ones_v)
    base = sid * RPT
    pltpu.sync_copy(zc_hbm, cnt_sh.at[pl.ds(base, RPT)])
    plsc.subcore_barrier()

    def chunk(j, c):
        pltpu.sync_copy(ones_v, cnt_sh.at[dst_v.at[j]], add=True)
        return c

    lax.fori_loop(0, NCH, chunk, 0)

    plsc.subcore_barrier()
    pltpu.sync_copy(cnt_sh.at[pl.ds(base, RPT)],
                    cnts_hbm.at[cid, pl.ds(base, RPT)])


_sc_count = functools.partial(
    pl.kernel,
    out_type=jax.ShapeDtypeStruct((NC, NP, CW), jnp.float32),
    mesh=plsc.VectorSubcoreMesh(core_axis_name="c", subcore_axis_name="s"),
    compiler_params=pltpu.CompilerParams(use_tc_tiling_on_sc=False),
    scratch_types=(
        pltpu.VMEM((NCH, K), jnp.int32),
        pltpu.VMEM((K, CW), jnp.float32),
        pltpu.VMEM_SHARED((NP, CW), jnp.float32),
    ),
)(_sc_count_body)


def _tc_body(x_ref, p0_ref, p1_ref, c0_ref, c1_ref, wl_ref, wr_ref, b_ref,
             o_ref):
    x = x_ref[...]
    s = p0_ref[...] + p1_ref[...]
    c = jnp.maximum(c0_ref[...][:, 0:1] + c1_ref[...][:, 0:1], 1.0)
    agg = s / c
    y = (jnp.dot(agg, wl_ref[...], preferred_element_type=jnp.float32)
         + b_ref[...]
         + jnp.dot(x, wr_ref[...], preferred_element_type=jnp.float32))
    o_ref[...] = x + jnp.maximum(y, 0.0)


BT = 1000


def _tc_combine(x, p0, p1, c0, c1, wlT, wrT, b):
    grid = (N // BT,)
    return pl.pallas_call(
        _tc_body,
        out_shape=jax.ShapeDtypeStruct((N, D), jnp.float32),
        grid=grid,
        in_specs=[
            pl.BlockSpec((BT, D), lambda i: (i, 0)),
            pl.BlockSpec((BT, D), lambda i: (i, 0)),
            pl.BlockSpec((BT, D), lambda i: (i, 0)),
            pl.BlockSpec((BT, CW), lambda i: (i, 0)),
            pl.BlockSpec((BT, CW), lambda i: (i, 0)),
            pl.BlockSpec((D, D), lambda i: (0, 0)),
            pl.BlockSpec((D, D), lambda i: (0, 0)),
            pl.BlockSpec((1, D), lambda i: (0, 0)),
        ],
        out_specs=pl.BlockSpec((BT, D), lambda i: (i, 0)),
    )(x, p0, p1, c0, c1, wlT, wrT, b)


def kernel(x, edge_index, W_l, b_l, W_r):
    pad = EP - E
    src = jnp.concatenate(
        [edge_index[0], jnp.zeros((pad,), jnp.int32)]).reshape(NW, NCH, K)
    dst = jnp.concatenate(
        [edge_index[1], jnp.full((pad,), N, jnp.int32)]).reshape(NW, NCH, K)
    ones_col = jnp.ones((K, CW), jnp.float32)
    zeros_col = jnp.zeros((RPT, CW), jnp.float32)
    sums = _sc_aggregate(x, src, dst)
    cnts = _sc_count(dst, ones_col, zeros_col)
    return _tc_combine(x, sums[0], sums[1], cnts[0], cnts[1],
                       W_l.T, W_r.T, b_l.reshape(1, D))


# speedup vs baseline: 3.7709x; 1.1185x over previous
import functools

import jax
import jax.numpy as jnp
from jax import lax
from jax.experimental import pallas as pl
from jax.experimental.pallas import tpu as pltpu
from jax.experimental.pallas import tpu_sc as plsc

N = 10000
E = 320000
D = 128

NC = 2
NS = 16
NW = NC * NS
K = 64
NP = 10112
EP = 327680
NCH = EP // (NW * K)
RPT = NP // NS


CW = 8


NBUF = 2


def _sc_body(x_hbm, src_hbm, dst_hbm, sums_hbm,
             src_v, dst_v, rows0, rows1,
             acc_sh, gs0, gs1):
    cid = lax.axis_index("c")
    sid = lax.axis_index("s")
    wid = cid * NS + sid
    rows = (rows0, rows1)
    gsem = (gs0, gs1)

    pltpu.sync_copy(src_hbm.at[wid], src_v)
    pltpu.sync_copy(dst_hbm.at[wid], dst_v)

    zeros16 = jnp.zeros((16,), jnp.float32)

    def fill_rows(i, c):
        def inner(j, c2):
            rows0[i, pl.ds(j * 16, 16)] = zeros16
            return c2
        return lax.fori_loop(0, D // 16, inner, c)

    lax.fori_loop(0, K, fill_rows, 0)

    base = sid * RPT

    def zero_chunk(t, c):
        pltpu.sync_copy(rows0, acc_sh.at[pl.ds(base + t * K, K)])
        return c

    lax.fori_loop(0, RPT // K, zero_chunk, 0)
    if RPT % K:
        pltpu.sync_copy(rows0.at[pl.ds(0, RPT % K)],
                        acc_sh.at[pl.ds(base + (RPT // K) * K, RPT % K)])
    plsc.subcore_barrier()

    def gfire(b, j):
        pltpu.async_copy(x_hbm.at[src_v.at[j]], rows[b], gsem[b])

    def gwait(b, j):
        pltpu.make_async_copy(x_hbm.at[src_v.at[j]], rows[b], gsem[b]).wait()

    gfire(0, 0)

    def group(g, c):
        for b in range(NBUF):
            j = g * NBUF + b

            @pl.when(j + 1 < NCH)
            def _():
                gfire(1 - b, j + 1)

            gwait(b, j)
            pltpu.sync_copy(rows[b], acc_sh.at[dst_v.at[j]], add=True)
        return c

    lax.fori_loop(0, NCH // NBUF, group, 0)

    plsc.subcore_barrier()

    pltpu.sync_copy(acc_sh.at[pl.ds(base, RPT)],
                    sums_hbm.at[cid, pl.ds(base, RPT)])


_sc_aggregate = functools.partial(
    pl.kernel,
    out_type=jax.ShapeDtypeStruct((NC, NP, D), jnp.float32),
    mesh=plsc.VectorSubcoreMesh(core_axis_name="c", subcore_axis_name="s"),
    compiler_params=pltpu.CompilerParams(use_tc_tiling_on_sc=False),
    scratch_types=(
        pltpu.VMEM((NCH, K), jnp.int32),
        pltpu.VMEM((NCH, K), jnp.int32),
        pltpu.VMEM((K, D), jnp.float32),
        pltpu.VMEM((K, D), jnp.float32),
        pltpu.VMEM_SHARED((NP, D), jnp.float32),
        pltpu.SemaphoreType.DMA,
        pltpu.SemaphoreType.DMA,
    ),
)(_sc_body)


def _sc_count_body(dst_hbm, ones_hbm, zc_hbm, cnts_hbm,
                   dst_v, ones_v, cnt_sh):
    cid = lax.axis_index("c")
    sid = lax.axis_index("s")
    wid = cid * NS + sid

    pltpu.sync_copy(dst_hbm.at[wid], dst_v)
    pltpu.sync_copy(ones_hbm, ones_v)
    base = sid * RPT
    pltpu.sync_copy(zc_hbm, cnt_sh.at[pl.ds(base, RPT)])
    plsc.subcore_barrier()

    def chunk(j, c):
        pltpu.sync_copy(ones_v, cnt_sh.at[dst_v.at[j]], add=True)
        return c

    lax.fori_loop(0, NCH, chunk, 0)

    plsc.subcore_barrier()
    pltpu.sync_copy(cnt_sh.at[pl.ds(base, RPT)],
                    cnts_hbm.at[cid, pl.ds(base, RPT)])


_sc_count = functools.partial(
    pl.kernel,
    out_type=jax.ShapeDtypeStruct((NC, NP, CW), jnp.float32),
    mesh=plsc.VectorSubcoreMesh(core_axis_name="c", subcore_axis_name="s"),
    compiler_params=pltpu.CompilerParams(use_tc_tiling_on_sc=False),
    scratch_types=(
        pltpu.VMEM((NCH, K), jnp.int32),
        pltpu.VMEM((K, CW), jnp.float32),
        pltpu.VMEM_SHARED((NP, CW), jnp.float32),
    ),
)(_sc_count_body)


def _tc_body(x_ref, p0_ref, p1_ref, c0_ref, c1_ref, wl_ref, wr_ref, b_ref,
             o_ref):
    x = x_ref[...]
    s = p0_ref[...] + p1_ref[...]
    c = jnp.maximum(c0_ref[...][:, 0:1] + c1_ref[...][:, 0:1], 1.0)
    agg = s / c
    y = (jnp.dot(agg, wl_ref[...], preferred_element_type=jnp.float32)
         + b_ref[...]
         + jnp.dot(x, wr_ref[...], preferred_element_type=jnp.float32))
    o_ref[...] = x + jnp.maximum(y, 0.0)


BT = 1000


def _tc_combine(x, p0, p1, c0, c1, wlT, wrT, b):
    grid = (N // BT,)
    return pl.pallas_call(
        _tc_body,
        out_shape=jax.ShapeDtypeStruct((N, D), jnp.float32),
        grid=grid,
        in_specs=[
            pl.BlockSpec((BT, D), lambda i: (i, 0)),
            pl.BlockSpec((BT, D), lambda i: (i, 0)),
            pl.BlockSpec((BT, D), lambda i: (i, 0)),
            pl.BlockSpec((BT, CW), lambda i: (i, 0)),
            pl.BlockSpec((BT, CW), lambda i: (i, 0)),
            pl.BlockSpec((D, D), lambda i: (0, 0)),
            pl.BlockSpec((D, D), lambda i: (0, 0)),
            pl.BlockSpec((1, D), lambda i: (0, 0)),
        ],
        out_specs=pl.BlockSpec((BT, D), lambda i: (i, 0)),
    )(x, p0, p1, c0, c1, wlT, wrT, b)


def kernel(x, edge_index, W_l, b_l, W_r):
    pad = EP - E
    src = jnp.concatenate(
        [edge_index[0], jnp.zeros((pad,), jnp.int32)]).reshape(NW, NCH, K)
    dst = jnp.concatenate(
        [edge_index[1], jnp.full((pad,), N, jnp.int32)]).reshape(NW, NCH, K)
    ones_col = jnp.ones((K, CW), jnp.float32)
    zeros_col = jnp.zeros((RPT, CW), jnp.float32)
    sums = _sc_aggregate(x, src, dst)
    cnts = _sc_count(dst, ones_col, zeros_col)
    return _tc_combine(x, sums[0], sums[1], cnts[0], cnts[1],
                       W_l.T, W_r.T, b_l.reshape(1, D))
